# Initial kernel scaffold; baseline (speedup 1.0000x reference)
import jax
import jax.numpy as jnp
from jax.experimental import pallas as pl


def kernel(x, edge_index, batch_vector, W1, b1, W2, b2, Wl, bl):
    raise NotImplementedError("write your pallas kernel here")


# speedup vs baseline: 47.3108x; 47.3108x over previous
import functools

import jax
import jax.numpy as jnp
from jax import lax
from jax.experimental import pallas as pl
from jax.experimental.pallas import tpu as pltpu
from jax.experimental.pallas import tpu_sc as plsc

N = 100000
E = 3200000
IN_CH = 16
HID = 32
NUM_CLASSES = 2
NUM_GRAPHS = 1024

NC = 2
NS = 16
NW = NC * NS
LANE = 128
CHUNK_R = 8

CHUNKS_L1 = 98
EP = NW * CHUNKS_L1 * CHUNK_R * LANE
RT = EP // LANE
ROWS_T_L1 = RT // NW
CHUNKS_L2 = 2 * CHUNKS_L1
ROWS_T_L2 = RT // NS

TRASH = 352
N_ACC = N + TRASH
ROWS_TILE_ACC = N_ACC // NS

_MESH = plsc.VectorSubcoreMesh(core_axis_name="c", subcore_axis_name="s")
_SC_PARAMS = pltpu.CompilerParams(use_tc_tiling_on_sc=False)


def _zero_fill_rows(zbuf, nrows, ncols):
  for r in range(nrows):
    zbuf[r, :] = jnp.zeros((ncols,), jnp.float32)


@functools.partial(
    pl.kernel,
    out_type=jax.ShapeDtypeStruct((NC * N_ACC,), jnp.float32),
    mesh=_MESH,
    compiler_params=_SC_PARAMS,
    scratch_types=[
        pltpu.VMEM_SHARED((N_ACC,), jnp.float32),
        pltpu.VMEM((CHUNK_R, LANE), jnp.int32),
        pltpu.VMEM((1568,), jnp.float32),
        pltpu.VMEM((LANE,), jnp.float32),
        pltpu.SemaphoreType.DMA,
    ],
)
def _deg_kernel(dst_hbm, out_hbm, acc, didx, tmp, ones, sem):
  cid = lax.axis_index("c")
  sid = lax.axis_index("s")
  for i in range(LANE // 16):
    ones[pl.ds(i * 16, 16)] = jnp.ones((16,), jnp.float32)
  for i in range(1568 // 16):
    tmp[pl.ds(i * 16, 16)] = jnp.zeros((16,), jnp.float32)
  base = sid * ROWS_TILE_ACC
  zds = [
      pltpu.async_copy(tmp, acc.at[pl.ds(base + i * 1568, 1568)], sem)
      for i in range(ROWS_TILE_ACC // 1568)
  ]
  for d in zds:
    d.wait()
  plsc.subcore_barrier()

  wid = cid * NS + sid
  row0 = wid * ROWS_T_L1

  def body(c, carry):
    pltpu.sync_copy(dst_hbm.at[pl.ds(row0 + c * CHUNK_R, CHUNK_R)], didx)
    ds_ = [
        pltpu.async_copy(ones, acc.at[didx.at[j]], sem, add=True)
        for j in range(CHUNK_R)
    ]
    for d in ds_:
      d.wait()
    return carry

  lax.fori_loop(0, CHUNKS_L1, body, 0)
  plsc.subcore_barrier()
  for i in range(ROWS_TILE_ACC // 1568):
    pltpu.sync_copy(acc.at[pl.ds(base + i * 1568, 1568)], tmp)
    pltpu.sync_copy(tmp, out_hbm.at[pl.ds(cid * N_ACC + base + i * 1568, 1568)])


def _agg_body(g_hbm, src_hbm, dst_hbm, out_hbm, acc, sidx, didx, rows, zbuf,
              gsem, split_edges):
  cid = lax.axis_index("c")
  sid = lax.axis_index("s")
  _zero_fill_rows(zbuf, LANE, IN_CH)
  base = sid * ROWS_TILE_ACC
  zds = [
      pltpu.async_copy(zbuf, acc.at[pl.ds(base + i * LANE, LANE)], gsem)
      for i in range(ROWS_TILE_ACC // LANE)
  ]
  for d in zds:
    d.wait()
  plsc.subcore_barrier()

  if split_edges:
    row0 = (cid * NS + sid) * ROWS_T_L1
    nchunks = CHUNKS_L1
    src_sel = 0
  else:
    row0 = sid * ROWS_T_L2
    nchunks = CHUNKS_L2
    src_sel = cid

  def body(c, carry):
    r = row0 + c * CHUNK_R
    pltpu.sync_copy(src_hbm.at[src_sel, pl.ds(r, CHUNK_R)], sidx)
    pltpu.sync_copy(dst_hbm.at[pl.ds(r, CHUNK_R)], didx)
    gds = [
        pltpu.async_copy(g_hbm.at[sidx.at[j]], rows.at[j], gsem)
        for j in range(CHUNK_R)
    ]
    for d in gds:
      d.wait()
    sds = [
        pltpu.async_copy(rows.at[j], acc.at[didx.at[j]], gsem, add=True)
        for j in range(CHUNK_R)
    ]
    for d in sds:
      d.wait()
    return carry

  lax.fori_loop(0, nchunks, body, 0)
  plsc.subcore_barrier()
  for i in range(ROWS_TILE_ACC // LANE):
    pltpu.sync_copy(acc.at[pl.ds(base + i * LANE, LANE)], zbuf)
    pltpu.sync_copy(zbuf, out_hbm.at[cid, pl.ds(base + i * LANE, LANE)])


_AGG_SCRATCH = [
    pltpu.VMEM_SHARED((N_ACC, IN_CH), jnp.float32),
    pltpu.VMEM((CHUNK_R, LANE), jnp.int32),
    pltpu.VMEM((CHUNK_R, LANE), jnp.int32),
    pltpu.VMEM((CHUNK_R, LANE, IN_CH), jnp.float32),
    pltpu.VMEM((LANE, IN_CH), jnp.float32),
    pltpu.SemaphoreType.DMA,
]

_l1_kernel = functools.partial(
    pl.kernel,
    out_type=jax.ShapeDtypeStruct((NC, N_ACC, IN_CH), jnp.float32),
    mesh=_MESH,
    compiler_params=_SC_PARAMS,
    scratch_types=_AGG_SCRATCH,
)(functools.partial(_agg_body, split_edges=True))

_l2_kernel = functools.partial(
    pl.kernel,
    out_type=jax.ShapeDtypeStruct((NC, N_ACC, IN_CH), jnp.float32),
    mesh=_MESH,
    compiler_params=_SC_PARAMS,
    scratch_types=_AGG_SCRATCH,
)(functools.partial(_agg_body, split_edges=False))


_B1 = 2048
_GRID1 = (N + _B1 - 1) // _B1


def _tc1_body(deg0, deg1, x, dinv_ref, gx_ref):
  d = deg0[...] + deg1[...] + 1.0
  dv = lax.rsqrt(d)
  dinv_ref[...] = dv
  gx_ref[...] = x[...] * dv[:, None]


def _tc1(deg0, deg1, x):
  grid = _GRID1
  return pl.pallas_call(
      _tc1_body,
      grid=(grid,),
      in_specs=[
          pl.BlockSpec((_B1,), lambda i: (i,)),
          pl.BlockSpec((_B1,), lambda i: (i,)),
          pl.BlockSpec((_B1, IN_CH), lambda i: (i, 0)),
      ],
      out_specs=[
          pl.BlockSpec((_B1,), lambda i: (i,)),
          pl.BlockSpec((_B1, IN_CH), lambda i: (i, 0)),
      ],
      out_shape=[
          jax.ShapeDtypeStruct((N,), jnp.float32),
          jax.ShapeDtypeStruct((N, IN_CH), jnp.float32),
      ],
  )(deg0, deg1, x)


def _tcmid_body(p1, gx, dinv, w1, b1, q_ref):
  agg = p1[0] + p1[1] + gx[...]
  dv = dinv[...]
  u = agg * dv[:, None]
  h = jnp.dot(u, w1[...], preferred_element_type=jnp.float32) + b1[...][None, :]
  q = jnp.maximum(h, 0.0) * dv[:, None]
  q_ref[0] = q[:, :IN_CH]
  q_ref[1] = q[:, IN_CH:]


def _tcmid(p1, gx, dinv, w1, b1):
  grid = _GRID1
  return pl.pallas_call(
      _tcmid_body,
      grid=(grid,),
      in_specs=[
          pl.BlockSpec((2, _B1, IN_CH), lambda i: (0, i, 0)),
          pl.BlockSpec((_B1, IN_CH), lambda i: (i, 0)),
          pl.BlockSpec((_B1,), lambda i: (i,)),
          pl.BlockSpec((IN_CH, HID), lambda i: (0, 0)),
          pl.BlockSpec((HID,), lambda i: (0,)),
      ],
      out_specs=[pl.BlockSpec((2, _B1, IN_CH), lambda i: (0, i, 0))],
      out_shape=[jax.ShapeDtypeStruct((2, N, IN_CH), jnp.float32)],
  )(p1, gx, dinv, w1, b1)[0]


_B2 = 1024
_GRID2 = (N + _B2 - 1) // _B2


def _tcfin_body(p2, q, dinv, w2, b2, batch, wl, bl, out_ref, sums, cnts):
  i = pl.program_id(0)

  @pl.when(i == 0)
  def _init():
    sums[...] = jnp.zeros((NUM_GRAPHS, HID), jnp.float32)
    cnts[...] = jnp.zeros((NUM_GRAPHS, 1), jnp.float32)

  agg = jnp.concatenate([p2[0] + q[0], p2[1] + q[1]], axis=1)
  dv = dinv[...]
  u = agg * dv[:, None]
  h = jnp.dot(u, w2[...], preferred_element_type=jnp.float32) + b2[...][None, :]
  r2 = jnp.maximum(h, 0.0)

  iot = lax.broadcasted_iota(jnp.int32, (NUM_GRAPHS, _B2), 0)
  col = lax.broadcasted_iota(jnp.int32, (NUM_GRAPHS, _B2), 1) + i * _B2
  oh = ((iot == batch[...][None, :]) & (col < N)).astype(jnp.float32)
  sums[...] += lax.dot_general(
      oh, r2, (((1,), (0,)), ((), ())), preferred_element_type=jnp.float32)
  cnts[...] += jnp.sum(oh, axis=1, keepdims=True)

  @pl.when(i == _GRID2 - 1)
  def _fin():
    pooled = sums[...] / jnp.maximum(cnts[...], 1.0)
    out_ref[...] = (
        jnp.dot(pooled, wl[...], preferred_element_type=jnp.float32)
        + bl[...][None, :])


def _tcfin(p2, q, dinv, w2, b2, batch_r, wl, bl):
  grid = _GRID2
  return pl.pallas_call(
      _tcfin_body,
      grid=(grid,),
      in_specs=[
          pl.BlockSpec((2, _B2, IN_CH), lambda i: (0, i, 0)),
          pl.BlockSpec((2, _B2, IN_CH), lambda i: (0, i, 0)),
          pl.BlockSpec((_B2,), lambda i: (i,)),
          pl.BlockSpec((HID, HID), lambda i: (0, 0)),
          pl.BlockSpec((HID,), lambda i: (0,)),
          pl.BlockSpec((_B2,), lambda i: (i,)),
          pl.BlockSpec((HID, NUM_CLASSES), lambda i: (0, 0)),
          pl.BlockSpec((NUM_CLASSES,), lambda i: (0,)),
      ],
      out_specs=[pl.BlockSpec((NUM_GRAPHS, NUM_CLASSES), lambda i: (0, 0))],
      out_shape=[jax.ShapeDtypeStruct((NUM_GRAPHS, NUM_CLASSES), jnp.float32)],
      scratch_shapes=[
          pltpu.VMEM((NUM_GRAPHS, HID), jnp.float32),
          pltpu.VMEM((NUM_GRAPHS, 1), jnp.float32),
      ],
  )(p2, q, dinv, w2, b2, batch_r, wl, bl)[0]


def kernel(x, edge_index, batch_vector, W1, b1, W2, b2, Wl, bl):
  src = edge_index[0]
  dst = edge_index[1]
  pad = EP - E
  ar = jnp.arange(pad, dtype=jnp.int32)
  srcp = jnp.concatenate([src, ar % N]).reshape(RT, LANE)
  dstp = jnp.concatenate([dst, N + (ar % TRASH)]).reshape(RT, LANE)
  src2 = jnp.stack([srcp, srcp + N])

  degp = _deg_kernel(dstp)
  dinv, gx = _tc1(degp[:N], degp[N_ACC:N_ACC + N], x)
  p1 = _l1_kernel(gx, src2, dstp)
  q = _tcmid(p1, gx, dinv, W1, b1)
  qf = q.reshape(2 * N, IN_CH)
  p2 = _l2_kernel(qf, src2, dstp)
  return _tcfin(p2, q, dinv, W2, b2, batch_vector, Wl, bl)


# speedup vs baseline: 60.5244x; 1.2793x over previous
import functools

import jax
import jax.numpy as jnp
from jax import lax
from jax.experimental import pallas as pl
from jax.experimental.pallas import tpu as pltpu
from jax.experimental.pallas import tpu_sc as plsc

N = 100000
E = 3200000
IN_CH = 16
HID = 32
NUM_CLASSES = 2
NUM_GRAPHS = 1024

NC = 2
NS = 16
NW = NC * NS
LANE = 128
CHUNK_R = 6

CHUNKS_L1 = 132
EP = NW * CHUNKS_L1 * CHUNK_R * LANE
RT = EP // LANE
ROWS_T_L1 = RT // NW
CHUNKS_L2 = 2 * CHUNKS_L1
ROWS_T_L2 = RT // NS

TRASH = 352
N_ACC = N + TRASH
ROWS_TILE_ACC = N_ACC // NS

_MESH = plsc.VectorSubcoreMesh(core_axis_name="c", subcore_axis_name="s")
_SC_PARAMS = pltpu.CompilerParams(use_tc_tiling_on_sc=False)


def _zero_fill_rows(zbuf, nrows, ncols):
  for r in range(nrows):
    zbuf[r, :] = jnp.zeros((ncols,), jnp.float32)


@functools.partial(
    pl.kernel,
    out_type=jax.ShapeDtypeStruct((NC * N_ACC,), jnp.float32),
    mesh=_MESH,
    compiler_params=_SC_PARAMS,
    scratch_types=[
        pltpu.VMEM_SHARED((N_ACC,), jnp.float32),
        pltpu.VMEM((CHUNK_R, LANE), jnp.int32),
        pltpu.VMEM((CHUNK_R, LANE), jnp.int32),
        pltpu.VMEM((1568,), jnp.float32),
        pltpu.VMEM((LANE,), jnp.float32),
        pltpu.SemaphoreType.DMA,
    ],
)
def _deg_kernel(dst_hbm, out_hbm, acc, didx_a, didx_b, tmp, ones, sem):
  cid = lax.axis_index("c")
  sid = lax.axis_index("s")
  for i in range(LANE // 16):
    ones[pl.ds(i * 16, 16)] = jnp.ones((16,), jnp.float32)
  for i in range(1568 // 16):
    tmp[pl.ds(i * 16, 16)] = jnp.zeros((16,), jnp.float32)
  base = sid * ROWS_TILE_ACC
  zds = [
      pltpu.async_copy(tmp, acc.at[pl.ds(base + i * 1568, 1568)], sem)
      for i in range(ROWS_TILE_ACC // 1568)
  ]
  for d in zds:
    d.wait()
  plsc.subcore_barrier()

  wid = cid * NS + sid
  row0 = wid * ROWS_T_L1

  def load_idx(c, buf):
    pltpu.sync_copy(dst_hbm.at[pl.ds(row0 + c * CHUNK_R, CHUNK_R)], buf)

  def fire_scat(buf):
    return [
        pltpu.async_copy(ones, acc.at[buf.at[j]], sem, add=True)
        for j in range(CHUNK_R)
    ]

  load_idx(0, didx_a)

  def body(g, carry):
    c0 = 2 * g
    da = fire_scat(didx_a)
    load_idx(c0 + 1, didx_b)
    for d in da:
      d.wait()
    db = fire_scat(didx_b)
    load_idx(lax.rem(c0 + 2, CHUNKS_L1), didx_a)
    for d in db:
      d.wait()
    return carry

  lax.fori_loop(0, CHUNKS_L1 // 2, body, 0)
  plsc.subcore_barrier()
  for i in range(ROWS_TILE_ACC // 1568):
    pltpu.sync_copy(acc.at[pl.ds(base + i * 1568, 1568)], tmp)
    pltpu.sync_copy(tmp, out_hbm.at[pl.ds(cid * N_ACC + base + i * 1568, 1568)])


def _agg_body(g_hbm, src_hbm, dst_hbm, out_hbm, acc, sidx_a, didx_a, rows_a,
              sidx_b, didx_b, rows_b, zbuf, gsem_a, gsem_b, ssem, split_edges):
  cid = lax.axis_index("c")
  sid = lax.axis_index("s")
  _zero_fill_rows(zbuf, LANE, IN_CH)
  base = sid * ROWS_TILE_ACC
  zds = [
      pltpu.async_copy(zbuf, acc.at[pl.ds(base + i * LANE, LANE)], gsem_a)
      for i in range(ROWS_TILE_ACC // LANE)
  ]
  for d in zds:
    d.wait()
  plsc.subcore_barrier()

  if split_edges:
    row0 = (cid * NS + sid) * ROWS_T_L1
    nchunks = CHUNKS_L1
    table = g_hbm
  else:
    row0 = sid * ROWS_T_L2
    nchunks = CHUNKS_L2
    table = g_hbm.at[pl.ds(cid * N, N)]

  def fire(c, sidx, didx, rows, gsem):
    r = row0 + c * CHUNK_R
    pltpu.sync_copy(src_hbm.at[pl.ds(r, CHUNK_R)], sidx)
    pltpu.sync_copy(dst_hbm.at[pl.ds(r, CHUNK_R)], didx)
    for j in range(CHUNK_R):
      pltpu.async_copy(table.at[sidx.at[j]], rows.at[j], gsem)

  def drain_gathers(sidx, rows, gsem):
    for j in range(CHUNK_R):
      pltpu.make_async_copy(table.at[sidx.at[j]], rows.at[j], gsem).wait()

  def scatter(didx, rows):
    sds = [
        pltpu.async_copy(rows.at[j], acc.at[didx.at[j]], ssem, add=True)
        for j in range(CHUNK_R)
    ]
    for d in sds:
      d.wait()

  fire(0, sidx_a, didx_a, rows_a, gsem_a)

  def body(g, carry):
    c0 = 2 * g
    fire(c0 + 1, sidx_b, didx_b, rows_b, gsem_b)
    drain_gathers(sidx_a, rows_a, gsem_a)
    scatter(didx_a, rows_a)
    fire(lax.rem(c0 + 2, nchunks), sidx_a, didx_a, rows_a, gsem_a)
    drain_gathers(sidx_b, rows_b, gsem_b)
    scatter(didx_b, rows_b)
    return carry

  lax.fori_loop(0, nchunks // 2, body, 0)
  drain_gathers(sidx_a, rows_a, gsem_a)
  plsc.subcore_barrier()
  for i in range(ROWS_TILE_ACC // LANE):
    pltpu.sync_copy(acc.at[pl.ds(base + i * LANE, LANE)], zbuf)
    pltpu.sync_copy(zbuf, out_hbm.at[cid, pl.ds(base + i * LANE, LANE)])


_AGG_SCRATCH = [
    pltpu.VMEM_SHARED((N_ACC, IN_CH), jnp.float32),
    pltpu.VMEM((CHUNK_R, LANE), jnp.int32),
    pltpu.VMEM((CHUNK_R, LANE), jnp.int32),
    pltpu.VMEM((CHUNK_R, LANE, IN_CH), jnp.float32),
    pltpu.VMEM((CHUNK_R, LANE), jnp.int32),
    pltpu.VMEM((CHUNK_R, LANE), jnp.int32),
    pltpu.VMEM((CHUNK_R, LANE, IN_CH), jnp.float32),
    pltpu.VMEM((LANE, IN_CH), jnp.float32),
    pltpu.SemaphoreType.DMA,
    pltpu.SemaphoreType.DMA,
    pltpu.SemaphoreType.DMA,
]

_l1_kernel = functools.partial(
    pl.kernel,
    out_type=jax.ShapeDtypeStruct((NC, N_ACC, IN_CH), jnp.float32),
    mesh=_MESH,
    compiler_params=_SC_PARAMS,
    scratch_types=_AGG_SCRATCH,
)(functools.partial(_agg_body, split_edges=True))

_l2_kernel = functools.partial(
    pl.kernel,
    out_type=jax.ShapeDtypeStruct((NC, N_ACC, IN_CH), jnp.float32),
    mesh=_MESH,
    compiler_params=_SC_PARAMS,
    scratch_types=_AGG_SCRATCH,
)(functools.partial(_agg_body, split_edges=False))


_B1 = 2048
_GRID1 = (N + _B1 - 1) // _B1


def _tc1_body(deg0, deg1, x, dinv_ref, gx_ref):
  d = deg0[...] + deg1[...] + 1.0
  dv = lax.rsqrt(d)
  dinv_ref[...] = dv
  gx_ref[...] = x[...] * dv[:, None]


def _tc1(deg0, deg1, x):
  grid = _GRID1
  return pl.pallas_call(
      _tc1_body,
      grid=(grid,),
      in_specs=[
          pl.BlockSpec((_B1,), lambda i: (i,)),
          pl.BlockSpec((_B1,), lambda i: (i,)),
          pl.BlockSpec((_B1, IN_CH), lambda i: (i, 0)),
      ],
      out_specs=[
          pl.BlockSpec((_B1,), lambda i: (i,)),
          pl.BlockSpec((_B1, IN_CH), lambda i: (i, 0)),
      ],
      out_shape=[
          jax.ShapeDtypeStruct((N,), jnp.float32),
          jax.ShapeDtypeStruct((N, IN_CH), jnp.float32),
      ],
  )(deg0, deg1, x)


def _tcmid_body(p1, gx, dinv, w1, b1, q_ref):
  agg = p1[0] + p1[1] + gx[...]
  dv = dinv[...]
  u = agg * dv[:, None]
  h = jnp.dot(u, w1[...], preferred_element_type=jnp.float32) + b1[...][None, :]
  q = jnp.maximum(h, 0.0) * dv[:, None]
  q_ref[0] = q[:, :IN_CH]
  q_ref[1] = q[:, IN_CH:]


def _tcmid(p1, gx, dinv, w1, b1):
  grid = _GRID1
  return pl.pallas_call(
      _tcmid_body,
      grid=(grid,),
      in_specs=[
          pl.BlockSpec((2, _B1, IN_CH), lambda i: (0, i, 0)),
          pl.BlockSpec((_B1, IN_CH), lambda i: (i, 0)),
          pl.BlockSpec((_B1,), lambda i: (i,)),
          pl.BlockSpec((IN_CH, HID), lambda i: (0, 0)),
          pl.BlockSpec((HID,), lambda i: (0,)),
      ],
      out_specs=[pl.BlockSpec((2, _B1, IN_CH), lambda i: (0, i, 0))],
      out_shape=[jax.ShapeDtypeStruct((2, N, IN_CH), jnp.float32)],
  )(p1, gx, dinv, w1, b1)[0]


_B2 = 1024
_GRID2 = (N + _B2 - 1) // _B2


def _tcfin_body(p2, q, dinv, w2, b2, batch, wl, bl, out_ref, sums, cnts):
  i = pl.program_id(0)

  @pl.when(i == 0)
  def _init():
    sums[...] = jnp.zeros((NUM_GRAPHS, HID), jnp.float32)
    cnts[...] = jnp.zeros((NUM_GRAPHS, 1), jnp.float32)

  agg = jnp.concatenate([p2[0] + q[0], p2[1] + q[1]], axis=1)
  dv = dinv[...]
  u = agg * dv[:, None]
  h = jnp.dot(u, w2[...], preferred_element_type=jnp.float32) + b2[...][None, :]
  r2 = jnp.maximum(h, 0.0)

  iot = lax.broadcasted_iota(jnp.int32, (NUM_GRAPHS, _B2), 0)
  col = lax.broadcasted_iota(jnp.int32, (NUM_GRAPHS, _B2), 1) + i * _B2
  oh = ((iot == batch[...][None, :]) & (col < N)).astype(jnp.float32)
  sums[...] += lax.dot_general(
      oh, r2, (((1,), (0,)), ((), ())), preferred_element_type=jnp.float32)
  cnts[...] += jnp.sum(oh, axis=1, keepdims=True)

  @pl.when(i == _GRID2 - 1)
  def _fin():
    pooled = sums[...] / jnp.maximum(cnts[...], 1.0)
    out_ref[...] = (
        jnp.dot(pooled, wl[...], preferred_element_type=jnp.float32)
        + bl[...][None, :])


def _tcfin(p2, q, dinv, w2, b2, batch_r, wl, bl):
  grid = _GRID2
  return pl.pallas_call(
      _tcfin_body,
      grid=(grid,),
      in_specs=[
          pl.BlockSpec((2, _B2, IN_CH), lambda i: (0, i, 0)),
          pl.BlockSpec((2, _B2, IN_CH), lambda i: (0, i, 0)),
          pl.BlockSpec((_B2,), lambda i: (i,)),
          pl.BlockSpec((HID, HID), lambda i: (0, 0)),
          pl.BlockSpec((HID,), lambda i: (0,)),
          pl.BlockSpec((_B2,), lambda i: (i,)),
          pl.BlockSpec((HID, NUM_CLASSES), lambda i: (0, 0)),
          pl.BlockSpec((NUM_CLASSES,), lambda i: (0,)),
      ],
      out_specs=[pl.BlockSpec((NUM_GRAPHS, NUM_CLASSES), lambda i: (0, 0))],
      out_shape=[jax.ShapeDtypeStruct((NUM_GRAPHS, NUM_CLASSES), jnp.float32)],
      scratch_shapes=[
          pltpu.VMEM((NUM_GRAPHS, HID), jnp.float32),
          pltpu.VMEM((NUM_GRAPHS, 1), jnp.float32),
      ],
  )(p2, q, dinv, w2, b2, batch_r, wl, bl)[0]


def kernel(x, edge_index, batch_vector, W1, b1, W2, b2, Wl, bl):
  src = edge_index[0]
  dst = edge_index[1]
  pad = EP - E
  ar = jnp.arange(pad, dtype=jnp.int32)
  srcp = jnp.concatenate([src, ar % N]).reshape(RT, LANE)
  dstp = jnp.concatenate([dst, N + (ar % TRASH)]).reshape(RT, LANE)

  degp = _deg_kernel(dstp)
  dinv, gx = _tc1(degp[:N], degp[N_ACC:N_ACC + N], x)
  p1 = _l1_kernel(gx, srcp, dstp)
  q = _tcmid(p1, gx, dinv, W1, b1)
  qf = q.reshape(2 * N, IN_CH)
  p2 = _l2_kernel(qf, srcp, dstp)
  return _tcfin(p2, q, dinv, W2, b2, batch_vector, Wl, bl)


# speedup vs baseline: 78.5993x; 1.2986x over previous
import functools

import jax
import jax.numpy as jnp
from jax import lax
from jax.experimental import pallas as pl
from jax.experimental.pallas import tpu as pltpu
from jax.experimental.pallas import tpu_sc as plsc

N = 100000
E = 3200000
IN_CH = 16
HID = 32
NUM_CLASSES = 2
NUM_GRAPHS = 1024

NC = 2
NS = 16
NW = NC * NS
LANE = 128
CHUNK_R = 6

CHUNKS_L1 = 132
EP = NW * CHUNKS_L1 * CHUNK_R * LANE
RT = EP // LANE
ROWS_T_L1 = RT // NW
CHUNKS_L2 = 2 * CHUNKS_L1
ROWS_T_L2 = RT // NS

TRASH = 352
N_ACC = N + TRASH
ROWS_TILE_ACC = N_ACC // NS

_MESH = plsc.VectorSubcoreMesh(core_axis_name="c", subcore_axis_name="s")
_SC_PARAMS = pltpu.CompilerParams(use_tc_tiling_on_sc=False)


def _zero_fill_rows(zbuf, nrows, ncols):
  for r in range(nrows):
    zbuf[r, :] = jnp.zeros((ncols,), jnp.float32)


@functools.partial(
    pl.kernel,
    out_type=jax.ShapeDtypeStruct((NC * N_ACC,), jnp.float32),
    mesh=_MESH,
    compiler_params=_SC_PARAMS,
    scratch_types=[
        pltpu.VMEM_SHARED((N_ACC,), jnp.float32),
        pltpu.VMEM((CHUNK_R, LANE), jnp.int32),
        pltpu.VMEM((CHUNK_R, LANE), jnp.int32),
        pltpu.VMEM((1568,), jnp.float32),
        pltpu.VMEM((LANE,), jnp.float32),
        pltpu.SemaphoreType.DMA,
    ],
)
def _deg_kernel(dst_hbm, out_hbm, acc, didx_a, didx_b, tmp, ones, sem):
  cid = lax.axis_index("c")
  sid = lax.axis_index("s")
  for i in range(LANE // 16):
    ones[pl.ds(i * 16, 16)] = jnp.ones((16,), jnp.float32)
  for i in range(1568 // 16):
    tmp[pl.ds(i * 16, 16)] = jnp.zeros((16,), jnp.float32)
  base = sid * ROWS_TILE_ACC
  zds = [
      pltpu.async_copy(tmp, acc.at[pl.ds(base + i * 1568, 1568)], sem)
      for i in range(ROWS_TILE_ACC // 1568)
  ]
  for d in zds:
    d.wait()
  plsc.subcore_barrier()

  wid = cid * NS + sid
  row0 = wid * ROWS_T_L1

  def load_idx(c, buf):
    pltpu.sync_copy(dst_hbm.at[pl.ds(row0 + c * CHUNK_R, CHUNK_R)], buf)

  def fire_scat(buf):
    return [
        pltpu.async_copy(ones, acc.at[buf.at[j]], sem, add=True)
        for j in range(CHUNK_R)
    ]

  load_idx(0, didx_a)

  def body(g, carry):
    c0 = 2 * g
    da = fire_scat(didx_a)
    load_idx(c0 + 1, didx_b)
    for d in da:
      d.wait()
    db = fire_scat(didx_b)
    load_idx(lax.rem(c0 + 2, CHUNKS_L1), didx_a)
    for d in db:
      d.wait()
    return carry

  lax.fori_loop(0, CHUNKS_L1 // 2, body, 0)
  plsc.subcore_barrier()
  for i in range(ROWS_TILE_ACC // 1568):
    pltpu.sync_copy(acc.at[pl.ds(base + i * 1568, 1568)], tmp)
    pltpu.sync_copy(tmp, out_hbm.at[pl.ds(cid * N_ACC + base + i * 1568, 1568)])


def _agg_body(g_hbm, src_hbm, dst_hbm, out_hbm, acc, sidx_a, didx_a, rows_a,
              sidx_b, didx_b, rows_b, zbuf, gsem_a, gsem_b, ssem, split_edges):
  cid = lax.axis_index("c")
  sid = lax.axis_index("s")
  _zero_fill_rows(zbuf, LANE, IN_CH)
  base = sid * ROWS_TILE_ACC
  zds = [
      pltpu.async_copy(zbuf, acc.at[pl.ds(base + i * LANE, LANE)], gsem_a)
      for i in range(ROWS_TILE_ACC // LANE)
  ]
  for d in zds:
    d.wait()
  plsc.subcore_barrier()

  if split_edges:
    row0 = (cid * NS + sid) * ROWS_T_L1
    nchunks = CHUNKS_L1
    table = g_hbm
  else:
    row0 = sid * ROWS_T_L2
    nchunks = CHUNKS_L2
    table = g_hbm.at[pl.ds(cid * N_ACC, N_ACC)]

  def fire(c, sidx, didx, rows, gsem):
    r = row0 + c * CHUNK_R
    pltpu.sync_copy(src_hbm.at[pl.ds(r, CHUNK_R)], sidx)
    pltpu.sync_copy(dst_hbm.at[pl.ds(r, CHUNK_R)], didx)
    for j in range(CHUNK_R):
      pltpu.async_copy(table.at[sidx.at[j]], rows.at[j], gsem)

  def drain_gathers(sidx, rows, gsem):
    for j in range(CHUNK_R):
      pltpu.make_async_copy(table.at[sidx.at[j]], rows.at[j], gsem).wait()

  def scatter(didx, rows):
    sds = [
        pltpu.async_copy(rows.at[j], acc.at[didx.at[j]], ssem, add=True)
        for j in range(CHUNK_R)
    ]
    for d in sds:
      d.wait()

  fire(0, sidx_a, didx_a, rows_a, gsem_a)

  def body(g, carry):
    c0 = 2 * g
    fire(c0 + 1, sidx_b, didx_b, rows_b, gsem_b)
    drain_gathers(sidx_a, rows_a, gsem_a)
    scatter(didx_a, rows_a)
    fire(lax.rem(c0 + 2, nchunks), sidx_a, didx_a, rows_a, gsem_a)
    drain_gathers(sidx_b, rows_b, gsem_b)
    scatter(didx_b, rows_b)
    return carry

  lax.fori_loop(0, nchunks // 2, body, 0)
  drain_gathers(sidx_a, rows_a, gsem_a)
  plsc.subcore_barrier()
  for i in range(ROWS_TILE_ACC // LANE):
    pltpu.sync_copy(acc.at[pl.ds(base + i * LANE, LANE)], zbuf)
    pltpu.sync_copy(zbuf, out_hbm.at[cid, pl.ds(base + i * LANE, LANE)])


_AGG_SCRATCH = [
    pltpu.VMEM_SHARED((N_ACC, IN_CH), jnp.float32),
    pltpu.VMEM((CHUNK_R, LANE), jnp.int32),
    pltpu.VMEM((CHUNK_R, LANE), jnp.int32),
    pltpu.VMEM((CHUNK_R, LANE, IN_CH), jnp.float32),
    pltpu.VMEM((CHUNK_R, LANE), jnp.int32),
    pltpu.VMEM((CHUNK_R, LANE), jnp.int32),
    pltpu.VMEM((CHUNK_R, LANE, IN_CH), jnp.float32),
    pltpu.VMEM((LANE, IN_CH), jnp.float32),
    pltpu.SemaphoreType.DMA,
    pltpu.SemaphoreType.DMA,
    pltpu.SemaphoreType.DMA,
]

_l1_kernel = functools.partial(
    pl.kernel,
    out_type=jax.ShapeDtypeStruct((NC, N_ACC, IN_CH), jnp.float32),
    mesh=_MESH,
    compiler_params=_SC_PARAMS,
    scratch_types=_AGG_SCRATCH,
)(functools.partial(_agg_body, split_edges=True))

_l2_kernel = functools.partial(
    pl.kernel,
    out_type=jax.ShapeDtypeStruct((NC, N_ACC, IN_CH), jnp.float32),
    mesh=_MESH,
    compiler_params=_SC_PARAMS,
    scratch_types=_AGG_SCRATCH,
)(functools.partial(_agg_body, split_edges=False))


PR = N_ACC // 8
_BP = 256
_GRIDP = PR // _BP


def _tc1_body(deg0, deg1, x, s16, dv_ref, gx_ref):
  d = deg0[...] + deg1[...] + 1.0
  dv = lax.rsqrt(d)
  dv_ref[...] = dv
  dinv_p = jnp.dot(dv, s16[...], preferred_element_type=jnp.float32)
  gx_ref[...] = x[...] * dinv_p


def _tc1(deg0, deg1, x, s16):
  return pl.pallas_call(
      _tc1_body,
      grid=(_GRIDP,),
      in_specs=[
          pl.BlockSpec((_BP, 8), lambda i: (i, 0)),
          pl.BlockSpec((_BP, 8), lambda i: (i, 0)),
          pl.BlockSpec((_BP, 128), lambda i: (i, 0)),
          pl.BlockSpec((8, 128), lambda i: (0, 0)),
      ],
      out_specs=[
          pl.BlockSpec((_BP, 8), lambda i: (i, 0)),
          pl.BlockSpec((_BP, 128), lambda i: (i, 0)),
      ],
      out_shape=[
          jax.ShapeDtypeStruct((PR, 8), jnp.float32),
          jax.ShapeDtypeStruct((PR, 128), jnp.float32),
      ],
  )(deg0, deg1, x, s16)


def _tcmid_body(p1, gx, dv, w1b, b1b, s16, s32, pa, pb, q_ref):
  agg = p1[0] + p1[1] + gx[...]
  dvb = dv[...]
  u = agg * jnp.dot(dvb, s16[...], preferred_element_type=jnp.float32)
  h = jnp.dot(u, w1b[...], preferred_element_type=jnp.float32) + b1b[...][None, :]
  q = jnp.maximum(h, 0.0) * jnp.dot(dvb, s32[...],
                                    preferred_element_type=jnp.float32)
  q_ref[0] = jnp.dot(q, pa[...], preferred_element_type=jnp.float32)
  q_ref[1] = jnp.dot(q, pb[...], preferred_element_type=jnp.float32)


def _tcmid(p1, gx, dv, w1b, b1b, s16, s32, pa, pb):
  return pl.pallas_call(
      _tcmid_body,
      grid=(_GRIDP,),
      in_specs=[
          pl.BlockSpec((2, _BP, 128), lambda i: (0, i, 0)),
          pl.BlockSpec((_BP, 128), lambda i: (i, 0)),
          pl.BlockSpec((_BP, 8), lambda i: (i, 0)),
          pl.BlockSpec((128, 256), lambda i: (0, 0)),
          pl.BlockSpec((256,), lambda i: (0,)),
          pl.BlockSpec((8, 128), lambda i: (0, 0)),
          pl.BlockSpec((8, 256), lambda i: (0, 0)),
          pl.BlockSpec((256, 128), lambda i: (0, 0)),
          pl.BlockSpec((256, 128), lambda i: (0, 0)),
      ],
      out_specs=[pl.BlockSpec((2, _BP, 128), lambda i: (0, i, 0))],
      out_shape=[jax.ShapeDtypeStruct((2, PR, 128), jnp.float32)],
  )(p1, gx, dv, w1b, b1b, s16, s32, pa, pb)[0]


def _tcfin_body(p2, q, dv, w2b, b2b, s32, pat, pbt, batcht, wl, bl, out_ref,
                sums, cnts):
  i = pl.program_id(0)

  @pl.when(i == 0)
  def _init():
    sums[...] = jnp.zeros((NUM_GRAPHS, HID), jnp.float32)
    cnts[...] = jnp.zeros((NUM_GRAPHS, 1), jnp.float32)

  agg = (jnp.dot(p2[0] + q[0], pat[...], preferred_element_type=jnp.float32)
         + jnp.dot(p2[1] + q[1], pbt[...], preferred_element_type=jnp.float32))
  u = agg * jnp.dot(dv[...], s32[...], preferred_element_type=jnp.float32)
  h = jnp.dot(u, w2b[...], preferred_element_type=jnp.float32) + b2b[...][None, :]
  r2 = jnp.maximum(h, 0.0)

  iot = lax.broadcasted_iota(jnp.int32, (NUM_GRAPHS, _BP), 0)
  bt = batcht[...]
  s_acc = sums[...]
  c_acc = cnts[...]
  for k in range(8):
    oh = (iot == bt[k][None, :]).astype(jnp.float32)
    s_acc += lax.dot_general(
        oh, r2[:, 32 * k:32 * k + 32], (((1,), (0,)), ((), ())),
        preferred_element_type=jnp.float32)
    c_acc += jnp.sum(oh, axis=1, keepdims=True)
  sums[...] = s_acc
  cnts[...] = c_acc

  @pl.when(i == _GRIDP - 1)
  def _fin():
    pooled = sums[...] / jnp.maximum(cnts[...], 1.0)
    out_ref[...] = (
        jnp.dot(pooled, wl[...], preferred_element_type=jnp.float32)
        + bl[...][None, :])


def _tcfin(p2, q, dv, w2b, b2b, s32, pat, pbt, batcht, wl, bl):
  return pl.pallas_call(
      _tcfin_body,
      grid=(_GRIDP,),
      in_specs=[
          pl.BlockSpec((2, _BP, 128), lambda i: (0, i, 0)),
          pl.BlockSpec((2, _BP, 128), lambda i: (0, i, 0)),
          pl.BlockSpec((_BP, 8), lambda i: (i, 0)),
          pl.BlockSpec((256, 256), lambda i: (0, 0)),
          pl.BlockSpec((256,), lambda i: (0,)),
          pl.BlockSpec((8, 256), lambda i: (0, 0)),
          pl.BlockSpec((128, 256), lambda i: (0, 0)),
          pl.BlockSpec((128, 256), lambda i: (0, 0)),
          pl.BlockSpec((8, _BP), lambda i: (0, i)),
          pl.BlockSpec((HID, NUM_CLASSES), lambda i: (0, 0)),
          pl.BlockSpec((NUM_CLASSES,), lambda i: (0,)),
      ],
      out_specs=[pl.BlockSpec((NUM_GRAPHS, NUM_CLASSES), lambda i: (0, 0))],
      out_shape=[jax.ShapeDtypeStruct((NUM_GRAPHS, NUM_CLASSES), jnp.float32)],
      scratch_shapes=[
          pltpu.VMEM((NUM_GRAPHS, HID), jnp.float32),
          pltpu.VMEM((NUM_GRAPHS, 1), jnp.float32),
      ],
  )(p2, q, dv, w2b, b2b, s32, pat, pbt, batcht, wl, bl)[0]


def kernel(x, edge_index, batch_vector, W1, b1, W2, b2, Wl, bl):
  src = edge_index[0]
  dst = edge_index[1]
  pad = EP - E
  ar = jnp.arange(pad, dtype=jnp.int32)
  srcp = jnp.concatenate([src, ar % N]).reshape(RT, LANE)
  dstp = jnp.concatenate([dst, N + (ar % TRASH)]).reshape(RT, LANE)

  f32 = jnp.float32
  eye8 = jnp.eye(8, dtype=f32)
  s16 = jnp.kron(eye8, jnp.ones((1, IN_CH), f32))
  s32 = jnp.kron(eye8, jnp.ones((1, HID), f32))
  pa = jnp.kron(eye8, jnp.eye(HID, IN_CH, dtype=f32))
  pb = jnp.kron(eye8, jnp.eye(HID, IN_CH, k=-IN_CH, dtype=f32))
  w1b = jnp.kron(eye8, W1)
  w2b = jnp.kron(eye8, W2)
  b1b = jnp.tile(b1, 8)
  b2b = jnp.tile(b2, 8)
  xp = jnp.concatenate(
      [x.reshape(N // 8, 128), jnp.zeros((PR - N // 8, 128), f32)])
  batcht = jnp.concatenate(
      [batch_vector,
       jnp.full((N_ACC - N,), NUM_GRAPHS, jnp.int32)]).reshape(PR, 8).T

  degp = _deg_kernel(dstp)
  dva, gxp = _tc1(degp[:N_ACC].reshape(PR, 8),
                  degp[N_ACC:].reshape(PR, 8), xp, s16)
  gx_t = gxp.reshape(N_ACC, IN_CH)
  p1 = _l1_kernel(gx_t, srcp, dstp)
  p1p = p1.reshape(2, PR, 128)
  qp = _tcmid(p1p, gxp, dva, w1b, b1b, s16, s32, pa, pb)
  qf = qp.reshape(2 * N_ACC, IN_CH)
  p2 = _l2_kernel(qf, srcp, dstp)
  p2p = p2.reshape(2, PR, 128)
  return _tcfin(p2p, qp, dva, w2b, b2b, s32, pa.T, pb.T, batcht, Wl, bl)
